# baseline (speedup 1.0000x reference)
import functools
import math

import jax
import jax.numpy as jnp
from jax import lax
from jax.experimental import pallas as pl
from jax.experimental.pallas import tpu as pltpu
from jax.experimental.pallas import tpu_sc as plsc

_NC = 2
_NS = 16
_NW = _NC * _NS
_L = 16

_CHUNK = 128
_K = 8
_MACRO = _CHUNK * _K


@functools.lru_cache(maxsize=None)
def _build(N, V, D):
    b_per_w = N // _NW
    n_macro = b_per_w // _MACRO
    scale = jnp.float32(math.sqrt(D))
    mesh = plsc.VectorSubcoreMesh(
        core_axis_name="c", subcore_axis_name="s",
        num_cores=_NC, num_subcores=_NS)

    @functools.partial(
        pl.kernel,
        mesh=mesh,
        out_type=jax.ShapeDtypeStruct((N, D), jnp.float32),
        scratch_types=[
            pltpu.VMEM((_K, _CHUNK), jnp.int32),
            pltpu.VMEM((_MACRO, D), jnp.float32),
            pltpu.SemaphoreType.DMA,
        ],
        compiler_params=pltpu.CompilerParams(use_tc_tiling_on_sc=False),
    )
    def gather_scale(idx_hbm, table_hbm, out_hbm, idx_v, rows_v, sem):
        wid = lax.axis_index("s") * _NC + lax.axis_index("c")
        base = wid * b_per_w

        def macro_body(m, carry):
            row0 = pl.multiple_of(base + m * _MACRO, _MACRO)
            pltpu.sync_copy(
                idx_hbm.at[pl.ds(pl.multiple_of(row0 // _CHUNK, _K), _K)],
                idx_v)
            copies = []
            for j in range(_K):
                copies.append(pltpu.async_copy(
                    table_hbm.at[idx_v.at[j]],
                    rows_v.at[pl.ds(j * _CHUNK, _CHUNK)],
                    sem))
            for c in copies:
                c.wait()

            def scale_body(i, c2):
                r0 = i * 4
                for dr in range(4):
                    for j in range(D // _L):
                        sl = (r0 + dr, pl.ds(j * _L, _L))
                        rows_v[sl] = rows_v[sl] * scale
                return c2
            lax.fori_loop(0, _MACRO // 4, scale_body, 0)

            pltpu.sync_copy(rows_v, out_hbm.at[pl.ds(row0, _MACRO)])
            return carry

        lax.fori_loop(0, n_macro, macro_body, 0)

    return gather_scale


def kernel(tokens, weight):
    B, S = tokens.shape
    V, D = weight.shape
    N = B * S
    idx2d = tokens.astype(jnp.int32).reshape(N // _CHUNK, _CHUNK)
    out = _build(N, V, D)(idx2d, weight)
    return out.reshape(B, S, D)
